# baseline (speedup 1.0000x reference)
import functools

import jax
import jax.numpy as jnp
from jax import lax
from jax.experimental import pallas as pl
from jax.experimental.pallas import tpu as pltpu
from jax.experimental.pallas import tpu_sc as plsc

_CLASS_NUM = 100000
_FEATURE_NUM = 64
_BATCH = 16384
_LAMBDAS = 2.0

_NC = 2
_NS = 16
_NW = _NC * _NS
_BPW = _BATCH // _NW
_LROW = 128
_CNT_PER_TILE = _BATCH // _NS
_CPAD = 16 * 6272


def _body(feat_hbm, lbl_hbm, center_hbm, out_hbm,
          table, lbl_cnt, ones_v, zeros_v, lbl_my, cnt_my, inv_my,
          cent_v, feat_v, acc_v, sem_c, sem_f):
  c = lax.axis_index("c")
  s = lax.axis_index("s")
  wid = s * _NC + c
  lrow0 = wid * (_BPW // _LROW)

  pltpu.sync_copy(lbl_hbm.at[pl.ds(lrow0, _BPW // _LROW)], lbl_my)
  feat_dma = pltpu.async_copy(
      feat_hbm.at[pl.ds(wid * _BPW, _BPW)], feat_v, sem_f)
  cent_dmas = [
      pltpu.async_copy(center_hbm.at[lbl_my.at[j]],
                       cent_v.at[pl.ds(j * _LROW, _LROW)], sem_c)
      for j in range(_BPW // _LROW)
  ]

  def fill_zeros(i, _):
    zeros_v[pl.ds(i * 16, 16)] = jnp.zeros((16,), jnp.float32)
    return 0
  lax.fori_loop(0, _CPAD // _NS // 16, fill_zeros, 0)

  def fill_ones(i, _):
    ones_v[pl.ds(i * 16, 16)] = jnp.ones((16,), jnp.float32)
    return 0
  lax.fori_loop(0, _CNT_PER_TILE // 16, fill_ones, 0)

  pltpu.sync_copy(zeros_v, table.at[pl.ds(s * (_CPAD // _NS), _CPAD // _NS)])
  plsc.subcore_barrier()

  pltpu.sync_copy(lbl_hbm.at[pl.ds(s * (_CNT_PER_TILE // _LROW),
                                   _CNT_PER_TILE // _LROW)], lbl_cnt)
  for j in range(_CNT_PER_TILE // _LROW):
    pltpu.sync_copy(ones_v.at[pl.ds(j * _LROW, _LROW)],
                    table.at[lbl_cnt.at[j]], add=True)
  plsc.subcore_barrier()

  for j in range(_BPW // _LROW):
    pltpu.sync_copy(table.at[lbl_my.at[j]],
                    cnt_my.at[pl.ds(j * _LROW, _LROW)])

  def invert(i, _):
    v = cnt_my[pl.ds(i * 16, 16)]
    inv_my[pl.ds(i * 16, 16)] = 1.0 / v
    return 0
  lax.fori_loop(0, _BPW // 16, invert, 0)

  feat_dma.wait()
  for d in cent_dmas:
    d.wait()

  def group(g, acc):
    wv16 = inv_my[pl.ds(g * 16, 16)]
    for i in range(16):
      r = g * 16 + i
      wv = jnp.full((16,), wv16[i], jnp.float32)
      for q in range(_FEATURE_NUM // 16):
        f = feat_v[r, pl.ds(q * 16, 16)]
        cc = cent_v[r, pl.ds(q * 16, 16)]
        d = f - cc
        acc = acc + d * d * wv
    return acc

  acc = lax.fori_loop(0, _BPW // 16, group, jnp.zeros((16,), jnp.float32))
  acc_v[...] = acc
  pltpu.sync_copy(acc_v, out_hbm.at[wid])


@jax.jit
def kernel(feature, label, center):
  lbl2d = label.astype(jnp.int32).reshape(_BATCH // _LROW, _LROW)
  mesh = plsc.VectorSubcoreMesh(core_axis_name="c", subcore_axis_name="s")
  kern = pl.kernel(
      _body,
      out_type=jax.ShapeDtypeStruct((_NW, 16), jnp.float32),
      mesh=mesh,
      compiler_params=pltpu.CompilerParams(use_tc_tiling_on_sc=False),
      scratch_types=[
          pltpu.VMEM_SHARED((_CPAD,), jnp.float32),
          pltpu.VMEM((_CNT_PER_TILE // _LROW, _LROW), jnp.int32),
          pltpu.VMEM((_CNT_PER_TILE,), jnp.float32),
          pltpu.VMEM((_CPAD // _NS,), jnp.float32),
          pltpu.VMEM((_BPW // _LROW, _LROW), jnp.int32),
          pltpu.VMEM((_BPW,), jnp.float32),
          pltpu.VMEM((_BPW,), jnp.float32),
          pltpu.VMEM((_BPW, _FEATURE_NUM), jnp.float32),
          pltpu.VMEM((_BPW, _FEATURE_NUM), jnp.float32),
          pltpu.VMEM((16,), jnp.float32),
          pltpu.SemaphoreType.DMA,
          pltpu.SemaphoreType.DMA,
      ],
  )
  partials = kern(feature, lbl2d, center)
  return jnp.sum(partials) * (_LAMBDAS / 2.0 / _BATCH)
